# baseline (speedup 1.0000x reference)
import jax
import jax.numpy as jnp
from jax import lax
from jax.experimental import pallas as pl
from jax.experimental.pallas import tpu as pltpu
from jax.experimental.pallas import tpu_sc as plsc

BATCH = 16384
EMBED_DIM = 32
ROWS_PER_128 = 128 // EMBED_DIM
PACKED_ROWS = 1000000 // ROWS_PER_128
NUM_CORES = 2
NUM_SUBCORES = 16
NUM_WORKERS = NUM_CORES * NUM_SUBCORES
BPW = BATCH // NUM_WORKERS
CHUNK = 128
NCHUNK = BPW // CHUNK
LANES = 16
GPC = CHUNK // LANES


def _mf_body(uidx_hbm, iidx_hbm, uemb_hbm, iemb_hbm, ulin_hbm, ilin_hbm,
             out_hbm, uidx_v, iidx_v, urow_idx, irow_idx,
             urows_a, irows_a, urows_b, irows_b,
             ubias_v, ibias_v, out_v, bias_sem, sem_a, sem_b):
  wid = lax.axis_index("s") * NUM_CORES + lax.axis_index("c")

  pltpu.sync_copy(uidx_hbm.at[pl.ds(wid * BPW, BPW)], uidx_v)
  pltpu.sync_copy(iidx_hbm.at[pl.ds(wid * BPW, BPW)], iidx_v)

  iota16 = lax.iota(jnp.int32, 16)
  ubufs, ibufs, sems = (urows_a, urows_b), (irows_a, irows_b), (sem_a, sem_b)

  def prep_rows(c):
    for g in range(GPC):
      sl = pl.ds(c * CHUNK + g * LANES, LANES)
      urow_idx[sl] = uidx_v[sl] // ROWS_PER_128
      irow_idx[sl] = iidx_v[sl] // ROWS_PER_128

  def fire(c):
    p = c % 2
    rows = pl.ds(c * CHUNK, CHUNK)
    return (pltpu.async_copy(uemb_hbm.at[urow_idx.at[rows]], ubufs[p], sems[p]),
            pltpu.async_copy(iemb_hbm.at[irow_idx.at[rows]], ibufs[p], sems[p]))

  def compute(c):
    p = c % 2
    for g in range(GPC):
      sl = pl.ds(c * CHUNK + g * LANES, LANES)
      slot = g * LANES + iota16
      uoff = (uidx_v[sl] & (ROWS_PER_128 - 1)) * EMBED_DIM
      ioff = (iidx_v[sl] & (ROWS_PER_128 - 1)) * EMBED_DIM
      acc = ubias_v[sl] + ibias_v[sl]
      for d in range(EMBED_DIM):
        u = plsc.load_gather(ubufs[p], [slot, uoff + d])
        it = plsc.load_gather(ibufs[p], [slot, ioff + d])
        acc = acc + u * it
      out_v[sl] = 1.0 / (1.0 + jnp.exp(-acc))

  bias_copies = []
  for c in range(NCHUNK):
    rows = pl.ds(c * CHUNK, CHUNK)
    bias_copies.append(pltpu.async_copy(
        ulin_hbm.at[uidx_v.at[rows]], ubias_v.at[rows], bias_sem))
    bias_copies.append(pltpu.async_copy(
        ilin_hbm.at[iidx_v.at[rows]], ibias_v.at[rows], bias_sem))

  prep_rows(0)
  inflight = [fire(0)]
  prep_rows(1)
  inflight.append(fire(1))
  for cp in bias_copies:
    cp.wait()

  for c in range(NCHUNK):
    for cp in inflight[c]:
      cp.wait()
    compute(c)
    if c + 2 < NCHUNK:
      prep_rows(c + 2)
      inflight.append(fire(c + 2))

  pltpu.sync_copy(out_v, out_hbm.at[pl.ds(wid * BPW, BPW)])


@jax.jit
def _mf_call(uidx, iidx, uemb128, iemb128, ulin_flat, ilin_flat):
  mesh = plsc.VectorSubcoreMesh(core_axis_name="c", subcore_axis_name="s")
  fn = pl.kernel(
      _mf_body,
      out_type=jax.ShapeDtypeStruct((BATCH,), jnp.float32),
      mesh=mesh,
      scratch_types=[
          pltpu.VMEM((BPW,), jnp.int32),
          pltpu.VMEM((BPW,), jnp.int32),
          pltpu.VMEM((BPW,), jnp.int32),
          pltpu.VMEM((BPW,), jnp.int32),
          pltpu.VMEM((CHUNK, 128), jnp.float32),
          pltpu.VMEM((CHUNK, 128), jnp.float32),
          pltpu.VMEM((CHUNK, 128), jnp.float32),
          pltpu.VMEM((CHUNK, 128), jnp.float32),
          pltpu.VMEM((BPW,), jnp.float32),
          pltpu.VMEM((BPW,), jnp.float32),
          pltpu.VMEM((BPW,), jnp.float32),
          pltpu.SemaphoreType.DMA,
          pltpu.SemaphoreType.DMA,
          pltpu.SemaphoreType.DMA,
      ],
      compiler_params=pltpu.CompilerParams(needs_layout_passes=False),
  )
  return fn(uidx, iidx, uemb128, iemb128, ulin_flat, ilin_flat)


def kernel(user_tensor, item_tensor, user_embed, item_embed, user_lin,
           item_lin):
  uemb128 = user_embed.reshape(PACKED_ROWS, 128)
  iemb128 = item_embed.reshape(PACKED_ROWS, 128)
  return _mf_call(user_tensor.astype(jnp.int32),
                  item_tensor.astype(jnp.int32),
                  uemb128, iemb128,
                  user_lin.reshape(-1), item_lin.reshape(-1))
